# baseline (speedup 1.0000x reference)
import functools

import jax
import jax.numpy as jnp
from jax import lax
from jax.experimental import pallas as pl
from jax.experimental.pallas import tpu as pltpu
from jax.experimental.pallas import tpu_sc as plsc

N = 10000
E = 320000
DIN = 128
H = 256
C = 64

DC = 128

NPAD = 10240
ROWS_PER_TILE = NPAD // 16
ROW_CHUNK = 128
N_ROW_CHUNKS = ROWS_PER_TILE // ROW_CHUNK
ECHUNK = 128
NCHUNKS = 2560
EPAD = NCHUNKS * ECHUNK
NBUF = 2

BN = 2000


def _zero_acc(zrows, stage_v, acc, row0):
  pltpu.sync_copy(zrows, stage_v)
  for k in range(N_ROW_CHUNKS):
    pltpu.sync_copy(stage_v, acc.at[pl.ds(row0 + k * ROW_CHUNK, ROW_CHUNK)])


def _edge_loop(x_ref, srcp, dstp, src_v, dst_v, rows_v, acc, sem,
               chunk0, nchunks):
  def body(j, carry):
    e0 = (chunk0 + j) * ECHUNK
    pltpu.sync_copy(srcp.at[pl.ds(e0, ECHUNK)], src_v)
    pltpu.sync_copy(dstp.at[pl.ds(e0, ECHUNK)], dst_v)
    pltpu.async_copy(x_ref.at[src_v], rows_v, sem).wait()
    pltpu.sync_copy(rows_v, acc.at[dst_v], add=True)
    return carry

  lax.fori_loop(0, nchunks, body, 0)


def _write_acc(acc, stage_v, out_ref, row0):
  for k in range(N_ROW_CHUNKS):
    r = row0 + k * ROW_CHUNK
    pltpu.sync_copy(acc.at[pl.ds(r, ROW_CHUNK)], stage_v)
    pltpu.sync_copy(stage_v, out_ref.at[pl.ds(r, ROW_CHUNK)])


def _sc_scratch():
  return [
      pltpu.VMEM((ECHUNK,), jnp.int32),
      pltpu.VMEM((ECHUNK,), jnp.int32),
      pltpu.VMEM((ECHUNK, DC), jnp.float32),
      pltpu.VMEM((ROW_CHUNK, DC), jnp.float32),
      pltpu.VMEM_SHARED((NPAD, DC), jnp.float32),
      pltpu.SemaphoreType.DMA,
  ]


def _make_sc_scatter_edgesplit():
  mesh = plsc.VectorSubcoreMesh(core_axis_name="c", subcore_axis_name="s")

  @functools.partial(
      pl.kernel,
      out_type=(jax.ShapeDtypeStruct((NPAD, DC), jnp.float32),
                jax.ShapeDtypeStruct((NPAD, DC), jnp.float32)),
      mesh=mesh,
      scratch_types=_sc_scratch(),
  )
  def sc_kernel(x, srcp, dstp, zrows, out0, out1,
                src_v, dst_v, rows_v, stage_v, acc, sem):
    cid = lax.axis_index("c")
    sid = lax.axis_index("s")
    row0 = sid * ROWS_PER_TILE
    per_tile = NCHUNKS // 32

    _zero_acc(zrows, stage_v, acc, row0)
    plsc.subcore_barrier()
    chunk0 = (cid * 16 + sid) * per_tile
    _edge_loop(x, srcp, dstp, src_v, dst_v, rows_v, acc, sem,
               chunk0, per_tile)
    plsc.subcore_barrier()
    pl.when(cid == 0)(lambda: _write_acc(acc, stage_v, out0, row0))
    pl.when(cid == 1)(lambda: _write_acc(acc, stage_v, out1, row0))

  return sc_kernel


def _make_sc_scatter_colsplit():
  mesh = plsc.VectorSubcoreMesh(core_axis_name="c", subcore_axis_name="s")

  @functools.partial(
      pl.kernel,
      out_type=(jax.ShapeDtypeStruct((NPAD, DC), jnp.float32),
                jax.ShapeDtypeStruct((NPAD, DC), jnp.float32)),
      mesh=mesh,
      scratch_types=_sc_scratch(),
  )
  def sc_kernel(xlo, xhi, srcp, dstp, zrows, outlo, outhi,
                src_v, dst_v, rows_v, stage_v, acc, sem):
    cid = lax.axis_index("c")
    sid = lax.axis_index("s")
    row0 = sid * ROWS_PER_TILE
    per_tile = NCHUNKS // 16

    def run(x_ref, out_ref):
      _zero_acc(zrows, stage_v, acc, row0)
      plsc.subcore_barrier()
      _edge_loop(x_ref, srcp, dstp, src_v, dst_v, rows_v, acc, sem,
                 sid * per_tile, per_tile)
      plsc.subcore_barrier()
      _write_acc(acc, stage_v, out_ref, row0)

    pl.when(cid == 0)(lambda: run(xlo, outlo))
    pl.when(cid == 1)(lambda: run(xhi, outhi))

  return sc_kernel


_sc_edgesplit = _make_sc_scatter_edgesplit()
_sc_colsplit = _make_sc_scatter_colsplit()


def _gin_mlp(parts, combine, Wa, ba, Wb, bb, din):
  nin = len(parts)

  def body(*refs):
    Wa_ref, ba_ref, Wb_ref, bb_ref, m_ref, st_ref = refs[nin:]
    h = combine([r[...] for r in refs[:nin]])
    u = jnp.maximum(jnp.dot(h, Wa_ref[...]) + ba_ref[...], 0.0)
    m = jnp.dot(u, Wb_ref[...]) + bb_ref[...]
    m_ref[...] = m

    @pl.when(pl.program_id(0) == 0)
    def _():
      st_ref[...] = jnp.zeros_like(st_ref)

    st_ref[0:1, :] = st_ref[0:1, :] + jnp.sum(m, axis=0, keepdims=True)
    st_ref[1:2, :] = st_ref[1:2, :] + jnp.sum(m * m, axis=0, keepdims=True)

  in_specs = (
      [pl.BlockSpec((BN, p.shape[1]), lambda i: (i, 0)) for p in parts]
      + [pl.BlockSpec((din, H), lambda i: (0, 0)),
         pl.BlockSpec((1, H), lambda i: (0, 0)),
         pl.BlockSpec((H, H), lambda i: (0, 0)),
         pl.BlockSpec((1, H), lambda i: (0, 0))]
  )
  m, st = pl.pallas_call(
      body,
      grid=(N // BN,),
      in_specs=in_specs,
      out_specs=[pl.BlockSpec((BN, H), lambda i: (i, 0)),
                 pl.BlockSpec((8, H), lambda i: (0, 0))],
      out_shape=[jax.ShapeDtypeStruct((N, H), jnp.float32),
                 jax.ShapeDtypeStruct((8, H), jnp.float32)],
  )(*parts, Wa, ba.reshape(1, H), Wb, bb.reshape(1, H))
  return m, st


def _affine_from_stats(st, g, be):
  mean = st[0] / N
  var = st[1] / N - mean * mean
  scale = g / jnp.sqrt(var + 1e-5)
  shift = be - mean * scale
  return scale.reshape(1, H), shift.reshape(1, H)


def _affine_relu_split(m, scale, shift):
  def body(m_ref, sc_ref, sh_ref, lo_ref, hi_ref):
    y = jnp.maximum(m_ref[...] * sc_ref[...] + sh_ref[...], 0.0)
    lo_ref[...] = y[:, :H // 2]
    hi_ref[...] = y[:, H // 2:]

  return pl.pallas_call(
      body,
      grid=(N // BN,),
      in_specs=[pl.BlockSpec((BN, H), lambda i: (i, 0)),
                pl.BlockSpec((1, H), lambda i: (0, 0)),
                pl.BlockSpec((1, H), lambda i: (0, 0))],
      out_specs=[pl.BlockSpec((BN, H // 2), lambda i: (i, 0)),
                 pl.BlockSpec((BN, H // 2), lambda i: (i, 0))],
      out_shape=[jax.ShapeDtypeStruct((N, H // 2), jnp.float32),
                 jax.ShapeDtypeStruct((N, H // 2), jnp.float32)],
  )(m, scale, shift)


def _final_head(m, scale, shift, Wfc, bfc):
  def body(m_ref, sc_ref, sh_ref, W_ref, b_ref, o_ref):
    y = jnp.maximum(m_ref[...] * sc_ref[...] + sh_ref[...], 0.0)
    logits = jnp.dot(y, W_ref[...]) + b_ref[...]
    mx = jnp.max(logits, axis=1, keepdims=True)
    lse = jnp.log(jnp.sum(jnp.exp(logits - mx), axis=1, keepdims=True))
    o_ref[...] = logits - mx - lse

  return pl.pallas_call(
      body,
      grid=(N // BN,),
      in_specs=[pl.BlockSpec((BN, H), lambda i: (i, 0)),
                pl.BlockSpec((1, H), lambda i: (0, 0)),
                pl.BlockSpec((1, H), lambda i: (0, 0)),
                pl.BlockSpec((H, C), lambda i: (0, 0)),
                pl.BlockSpec((1, C), lambda i: (0, 0))],
      out_specs=pl.BlockSpec((BN, C), lambda i: (i, 0)),
      out_shape=jax.ShapeDtypeStruct((N, C), jnp.float32),
  )(m, scale, shift, Wfc, bfc.reshape(1, C))


def kernel(x, edge_index, W1a, b1a, W1b, b1b, g1, be1,
           W2a, b2a, W2b, b2b, g2, be2, Wfc, bfc):
  srcp = jnp.concatenate([edge_index[0], jnp.zeros((EPAD - E,), jnp.int32)])
  dstp = jnp.concatenate([edge_index[1], jnp.full((EPAD - E,), N, jnp.int32)])
  zrows = jnp.zeros((ROW_CHUNK, DC), jnp.float32)

  a10, a11 = _sc_edgesplit(x, srcp, dstp, zrows)
  m1, st1 = _gin_mlp(
      [x, a10, a11], lambda v: v[0] + v[1] + v[2],
      W1a, b1a, W1b, b1b, DIN)
  sc1, sh1 = _affine_from_stats(st1, g1, be1)
  y1lo, y1hi = _affine_relu_split(m1, sc1, sh1)

  a2lo, a2hi = _sc_colsplit(y1lo, y1hi, srcp, dstp, zrows)
  m2, st2 = _gin_mlp(
      [y1lo, y1hi, a2lo, a2hi],
      lambda v: jnp.concatenate([v[0] + v[2], v[1] + v[3]], axis=1),
      W2a, b2a, W2b, b2b, H)
  sc2, sh2 = _affine_from_stats(st2, g2, be2)

  return _final_head(m2, sc2, sh2, Wfc, bfc)


# speedup vs baseline: 1.3197x; 1.3197x over previous
import functools

import jax
import jax.numpy as jnp
from jax import lax
from jax.experimental import pallas as pl
from jax.experimental.pallas import tpu as pltpu
from jax.experimental.pallas import tpu_sc as plsc

N = 10000
E = 320000
DIN = 128
H = 256
C = 64

DC = 128

NPAD = 10240
ROWS_PER_TILE = NPAD // 16
ROW_CHUNK = 128
N_ROW_CHUNKS = ROWS_PER_TILE // ROW_CHUNK
ECHUNK = 128
NCHUNKS = 2528
EPAD = NCHUNKS * ECHUNK
NBUF = 2

BN = 2000


def _zero_acc(zrows, stage_v, acc, row0):
  pltpu.sync_copy(zrows, stage_v)
  for k in range(N_ROW_CHUNKS):
    pltpu.sync_copy(stage_v, acc.at[pl.ds(row0 + k * ROW_CHUNK, ROW_CHUNK)])


def _edge_loop(x_ref, srcp, dstp, src_v, dst_v, rows_v, acc, sem,
               chunk0, nchunks):
  def body(j, carry):
    e0 = (chunk0 + j) * ECHUNK
    pltpu.sync_copy(srcp.at[pl.ds(e0, ECHUNK)], src_v)
    pltpu.sync_copy(dstp.at[pl.ds(e0, ECHUNK)], dst_v)
    pltpu.async_copy(x_ref.at[src_v], rows_v, sem).wait()
    pltpu.sync_copy(rows_v, acc.at[dst_v], add=True)
    return carry

  lax.fori_loop(0, nchunks, body, 0)


def _write_acc(acc, stage_v, out_ref, row0):
  for k in range(N_ROW_CHUNKS):
    r = row0 + k * ROW_CHUNK
    pltpu.sync_copy(acc.at[pl.ds(r, ROW_CHUNK)], stage_v)
    pltpu.sync_copy(stage_v, out_ref.at[pl.ds(r, ROW_CHUNK)])


def _sc_scratch():
  return [
      pltpu.VMEM((ECHUNK,), jnp.int32),
      pltpu.VMEM((ECHUNK,), jnp.int32),
      pltpu.VMEM((ECHUNK, DC), jnp.float32),
      pltpu.VMEM((ROW_CHUNK, DC), jnp.float32),
      pltpu.VMEM_SHARED((NPAD, DC), jnp.float32),
      pltpu.SemaphoreType.DMA,
  ]


def _make_sc_scatter_edgesplit():
  mesh = plsc.VectorSubcoreMesh(core_axis_name="c", subcore_axis_name="s")

  @functools.partial(
      pl.kernel,
      out_type=(jax.ShapeDtypeStruct((NPAD, DC), jnp.float32),
                jax.ShapeDtypeStruct((NPAD, DC), jnp.float32)),
      mesh=mesh,
      scratch_types=_sc_scratch(),
  )
  def sc_kernel(x, srcp, dstp, zrows, out0, out1,
                src_v, dst_v, rows_v, stage_v, acc, sem):
    cid = lax.axis_index("c")
    sid = lax.axis_index("s")
    row0 = sid * ROWS_PER_TILE
    per_tile = NCHUNKS // 32

    _zero_acc(zrows, stage_v, acc, row0)
    plsc.subcore_barrier()
    chunk0 = (cid * 16 + sid) * per_tile
    _edge_loop(x, srcp, dstp, src_v, dst_v, rows_v, acc, sem,
               chunk0, per_tile)
    plsc.subcore_barrier()
    pl.when(cid == 0)(lambda: _write_acc(acc, stage_v, out0, row0))
    pl.when(cid == 1)(lambda: _write_acc(acc, stage_v, out1, row0))

  return sc_kernel


def _make_sc_scatter_colsplit():
  mesh = plsc.VectorSubcoreMesh(core_axis_name="c", subcore_axis_name="s")

  @functools.partial(
      pl.kernel,
      out_type=(jax.ShapeDtypeStruct((NPAD, DC), jnp.float32),
                jax.ShapeDtypeStruct((NPAD, DC), jnp.float32)),
      mesh=mesh,
      scratch_types=_sc_scratch(),
  )
  def sc_kernel(xlo, xhi, srcp, dstp, zrows, outlo, outhi,
                src_v, dst_v, rows_v, stage_v, acc, sem):
    cid = lax.axis_index("c")
    sid = lax.axis_index("s")
    row0 = sid * ROWS_PER_TILE
    per_tile = NCHUNKS // 16

    def run(x_ref, out_ref):
      _zero_acc(zrows, stage_v, acc, row0)
      plsc.subcore_barrier()
      _edge_loop(x_ref, srcp, dstp, src_v, dst_v, rows_v, acc, sem,
                 sid * per_tile, per_tile)
      plsc.subcore_barrier()
      _write_acc(acc, stage_v, out_ref, row0)

    pl.when(cid == 0)(lambda: run(xlo, outlo))
    pl.when(cid == 1)(lambda: run(xhi, outhi))

  return sc_kernel


_sc_edgesplit = _make_sc_scatter_edgesplit()
_sc_colsplit = _make_sc_scatter_colsplit()


def _gin_mlp(parts, combine, Wa, ba, Wb, bb, din):
  nin = len(parts)

  def body(*refs):
    Wa_ref, ba_ref, Wb_ref, bb_ref, m_ref, st_ref = refs[nin:]
    h = combine([r[...] for r in refs[:nin]])
    u = jnp.maximum(jnp.dot(h, Wa_ref[...]) + ba_ref[...], 0.0)
    m = jnp.dot(u, Wb_ref[...]) + bb_ref[...]
    m_ref[...] = m

    @pl.when(pl.program_id(0) == 0)
    def _():
      st_ref[...] = jnp.zeros_like(st_ref)

    st_ref[0:1, :] = st_ref[0:1, :] + jnp.sum(m, axis=0, keepdims=True)
    st_ref[1:2, :] = st_ref[1:2, :] + jnp.sum(m * m, axis=0, keepdims=True)

  in_specs = (
      [pl.BlockSpec((BN, p.shape[1]), lambda i: (i, 0)) for p in parts]
      + [pl.BlockSpec((din, H), lambda i: (0, 0)),
         pl.BlockSpec((1, H), lambda i: (0, 0)),
         pl.BlockSpec((H, H), lambda i: (0, 0)),
         pl.BlockSpec((1, H), lambda i: (0, 0))]
  )
  m, st = pl.pallas_call(
      body,
      grid=(N // BN,),
      in_specs=in_specs,
      out_specs=[pl.BlockSpec((BN, H), lambda i: (i, 0)),
                 pl.BlockSpec((8, H), lambda i: (0, 0))],
      out_shape=[jax.ShapeDtypeStruct((N, H), jnp.float32),
                 jax.ShapeDtypeStruct((8, H), jnp.float32)],
  )(*parts, Wa, ba.reshape(1, H), Wb, bb.reshape(1, H))
  return m, st


def _affine_from_stats(st, g, be):
  mean = st[0] / N
  var = st[1] / N - mean * mean
  scale = g / jnp.sqrt(var + 1e-5)
  shift = be - mean * scale
  return scale.reshape(1, H), shift.reshape(1, H)


def _affine_relu_split(m, scale, shift):
  def body(m_ref, sc_ref, sh_ref, lo_ref, hi_ref):
    y = jnp.maximum(m_ref[...] * sc_ref[...] + sh_ref[...], 0.0)
    lo_ref[...] = y[:, :H // 2]
    hi_ref[...] = y[:, H // 2:]

  return pl.pallas_call(
      body,
      grid=(N // BN,),
      in_specs=[pl.BlockSpec((BN, H), lambda i: (i, 0)),
                pl.BlockSpec((1, H), lambda i: (0, 0)),
                pl.BlockSpec((1, H), lambda i: (0, 0))],
      out_specs=[pl.BlockSpec((BN, H // 2), lambda i: (i, 0)),
                 pl.BlockSpec((BN, H // 2), lambda i: (i, 0))],
      out_shape=[jax.ShapeDtypeStruct((N, H // 2), jnp.float32),
                 jax.ShapeDtypeStruct((N, H // 2), jnp.float32)],
  )(m, scale, shift)


def _final_head(m, scale, shift, Wfc, bfc):
  def body(m_ref, sc_ref, sh_ref, W_ref, b_ref, o_ref):
    y = jnp.maximum(m_ref[...] * sc_ref[...] + sh_ref[...], 0.0)
    logits = jnp.dot(y, W_ref[...]) + b_ref[...]
    mx = jnp.max(logits, axis=1, keepdims=True)
    lse = jnp.log(jnp.sum(jnp.exp(logits - mx), axis=1, keepdims=True))
    o_ref[...] = logits - mx - lse

  return pl.pallas_call(
      body,
      grid=(N // BN,),
      in_specs=[pl.BlockSpec((BN, H), lambda i: (i, 0)),
                pl.BlockSpec((1, H), lambda i: (0, 0)),
                pl.BlockSpec((1, H), lambda i: (0, 0)),
                pl.BlockSpec((H, C), lambda i: (0, 0)),
                pl.BlockSpec((1, C), lambda i: (0, 0))],
      out_specs=pl.BlockSpec((BN, C), lambda i: (i, 0)),
      out_shape=jax.ShapeDtypeStruct((N, C), jnp.float32),
  )(m, scale, shift, Wfc, bfc.reshape(1, C))


def kernel(x, edge_index, W1a, b1a, W1b, b1b, g1, be1,
           W2a, b2a, W2b, b2b, g2, be2, Wfc, bfc):
  srcp = jnp.concatenate([edge_index[0], jnp.zeros((EPAD - E,), jnp.int32)])
  pad_dst = N + jnp.arange(EPAD - E, dtype=jnp.int32) % (NPAD - N)
  dstp = jnp.concatenate([edge_index[1], pad_dst])
  zrows = jnp.zeros((ROW_CHUNK, DC), jnp.float32)

  a10, a11 = _sc_edgesplit(x, srcp, dstp, zrows)
  m1, st1 = _gin_mlp(
      [x, a10, a11], lambda v: v[0] + v[1] + v[2],
      W1a, b1a, W1b, b1b, DIN)
  sc1, sh1 = _affine_from_stats(st1, g1, be1)
  y1lo, y1hi = _affine_relu_split(m1, sc1, sh1)

  a2lo, a2hi = _sc_colsplit(y1lo, y1hi, srcp, dstp, zrows)
  m2, st2 = _gin_mlp(
      [y1lo, y1hi, a2lo, a2hi],
      lambda v: jnp.concatenate([v[0] + v[2], v[1] + v[3]], axis=1),
      W2a, b2a, W2b, b2b, H)
  sc2, sh2 = _affine_from_stats(st2, g2, be2)

  return _final_head(m2, sc2, sh2, Wfc, bfc)


# speedup vs baseline: 1.6107x; 1.2205x over previous
import functools

import jax
import jax.numpy as jnp
from jax import lax
from jax.experimental import pallas as pl
from jax.experimental.pallas import tpu as pltpu
from jax.experimental.pallas import tpu_sc as plsc

N = 10000
E = 320000
DIN = 128
H = 256
C = 64

DC = 128

NPAD = 10240
ROWS_PER_TILE = NPAD // 16
ROW_CHUNK = 128
N_ROW_CHUNKS = ROWS_PER_TILE // ROW_CHUNK
ECHUNK = 128
NCHUNKS = 2528
EPAD = NCHUNKS * ECHUNK
NBUF = 2

BN = 2000


def _zero_acc(zrows, stage_v, acc, row0):
  pltpu.sync_copy(zrows, stage_v)
  for k in range(N_ROW_CHUNKS):
    pltpu.sync_copy(stage_v, acc.at[pl.ds(row0 + k * ROW_CHUNK, ROW_CHUNK)])


def _edge_loop(x_ref, srcp, dstp, src_a, dst_a, src_b, dst_b, rows_v, acc,
               isem_a, isem_b, gsem, chunk0, nchunks):
  def start_idx(bs, bd, sem, j):
    e0 = (chunk0 + j) * ECHUNK
    pltpu.async_copy(srcp.at[pl.ds(e0, ECHUNK)], bs, sem)
    pltpu.async_copy(dstp.at[pl.ds(e0, ECHUNK)], bd, sem)

  def wait_idx(bs, bd, sem, j):
    e0 = (chunk0 + j) * ECHUNK
    pltpu.make_async_copy(srcp.at[pl.ds(e0, ECHUNK)], bs, sem).wait()
    pltpu.make_async_copy(dstp.at[pl.ds(e0, ECHUNK)], bd, sem).wait()

  def process(bs, bd, prefetch, j):
    g = pltpu.async_copy(x_ref.at[bs], rows_v, gsem)
    prefetch()
    g.wait()
    pltpu.sync_copy(rows_v, acc.at[bd], add=True)

  start_idx(src_a, dst_a, isem_a, 0)

  def body(jo, carry):
    j = jo * 2
    wait_idx(src_a, dst_a, isem_a, j)
    process(src_a, dst_a,
            functools.partial(start_idx, src_b, dst_b, isem_b, j + 1), j)
    wait_idx(src_b, dst_b, isem_b, j + 1)

    def prefetch_next(j=j):
      pl.when(j + 2 < nchunks)(
          functools.partial(start_idx, src_a, dst_a, isem_a, j + 2))

    process(src_b, dst_b, prefetch_next, j + 1)
    return carry

  lax.fori_loop(0, nchunks // 2, body, 0)
  if nchunks % 2:
    j = nchunks - 1
    wait_idx(src_a, dst_a, isem_a, j)
    process(src_a, dst_a, lambda: None, j)


def _write_acc(acc, stage_v, out_ref, row0):
  for k in range(N_ROW_CHUNKS):
    r = row0 + k * ROW_CHUNK
    pltpu.sync_copy(acc.at[pl.ds(r, ROW_CHUNK)], stage_v)
    pltpu.sync_copy(stage_v, out_ref.at[pl.ds(r, ROW_CHUNK)])


def _sc_scratch():
  return [
      pltpu.VMEM((ECHUNK,), jnp.int32),
      pltpu.VMEM((ECHUNK,), jnp.int32),
      pltpu.VMEM((ECHUNK,), jnp.int32),
      pltpu.VMEM((ECHUNK,), jnp.int32),
      pltpu.VMEM((ECHUNK, DC), jnp.float32),
      pltpu.VMEM((ROW_CHUNK, DC), jnp.float32),
      pltpu.VMEM_SHARED((NPAD, DC), jnp.float32),
      pltpu.SemaphoreType.DMA,
      pltpu.SemaphoreType.DMA,
      pltpu.SemaphoreType.DMA,
  ]


def _make_sc_scatter_edgesplit():
  mesh = plsc.VectorSubcoreMesh(core_axis_name="c", subcore_axis_name="s")

  @functools.partial(
      pl.kernel,
      out_type=(jax.ShapeDtypeStruct((NPAD, DC), jnp.float32),
                jax.ShapeDtypeStruct((NPAD, DC), jnp.float32)),
      mesh=mesh,
      scratch_types=_sc_scratch(),
  )
  def sc_kernel(x, srcp, dstp, zrows, out0, out1,
                src_a, dst_a, src_b, dst_b, rows_v, stage_v, acc,
                isem_a, isem_b, gsem):
    cid = lax.axis_index("c")
    sid = lax.axis_index("s")
    row0 = sid * ROWS_PER_TILE
    per_tile = NCHUNKS // 32

    _zero_acc(zrows, stage_v, acc, row0)
    plsc.subcore_barrier()
    chunk0 = (cid * 16 + sid) * per_tile
    _edge_loop(x, srcp, dstp, src_a, dst_a, src_b, dst_b, rows_v, acc,
               isem_a, isem_b, gsem, chunk0, per_tile)
    plsc.subcore_barrier()
    pl.when(cid == 0)(lambda: _write_acc(acc, stage_v, out0, row0))
    pl.when(cid == 1)(lambda: _write_acc(acc, stage_v, out1, row0))

  return sc_kernel


def _make_sc_scatter_colsplit():
  mesh = plsc.VectorSubcoreMesh(core_axis_name="c", subcore_axis_name="s")

  @functools.partial(
      pl.kernel,
      out_type=(jax.ShapeDtypeStruct((NPAD, DC), jnp.float32),
                jax.ShapeDtypeStruct((NPAD, DC), jnp.float32)),
      mesh=mesh,
      scratch_types=_sc_scratch(),
  )
  def sc_kernel(xlo, xhi, srcp, dstp, zrows, outlo, outhi,
                src_a, dst_a, src_b, dst_b, rows_v, stage_v, acc,
                isem_a, isem_b, gsem):
    cid = lax.axis_index("c")
    sid = lax.axis_index("s")
    row0 = sid * ROWS_PER_TILE
    per_tile = NCHUNKS // 16

    def run(x_ref, out_ref):
      _zero_acc(zrows, stage_v, acc, row0)
      plsc.subcore_barrier()
      _edge_loop(x_ref, srcp, dstp, src_a, dst_a, src_b, dst_b, rows_v,
                 acc, isem_a, isem_b, gsem, sid * per_tile, per_tile)
      plsc.subcore_barrier()
      _write_acc(acc, stage_v, out_ref, row0)

    pl.when(cid == 0)(lambda: run(xlo, outlo))
    pl.when(cid == 1)(lambda: run(xhi, outhi))

  return sc_kernel


_sc_edgesplit = _make_sc_scatter_edgesplit()
_sc_colsplit = _make_sc_scatter_colsplit()


def _gin_mlp(parts, combine, Wa, ba, Wb, bb, din):
  nin = len(parts)

  def body(*refs):
    Wa_ref, ba_ref, Wb_ref, bb_ref, m_ref, st_ref = refs[nin:]
    h = combine([r[...] for r in refs[:nin]])
    u = jnp.maximum(jnp.dot(h, Wa_ref[...]) + ba_ref[...], 0.0)
    m = jnp.dot(u, Wb_ref[...]) + bb_ref[...]
    m_ref[...] = m

    @pl.when(pl.program_id(0) == 0)
    def _():
      st_ref[...] = jnp.zeros_like(st_ref)

    st_ref[0:1, :] = st_ref[0:1, :] + jnp.sum(m, axis=0, keepdims=True)
    st_ref[1:2, :] = st_ref[1:2, :] + jnp.sum(m * m, axis=0, keepdims=True)

  in_specs = (
      [pl.BlockSpec((BN, p.shape[1]), lambda i: (i, 0)) for p in parts]
      + [pl.BlockSpec((din, H), lambda i: (0, 0)),
         pl.BlockSpec((1, H), lambda i: (0, 0)),
         pl.BlockSpec((H, H), lambda i: (0, 0)),
         pl.BlockSpec((1, H), lambda i: (0, 0))]
  )
  m, st = pl.pallas_call(
      body,
      grid=(N // BN,),
      in_specs=in_specs,
      out_specs=[pl.BlockSpec((BN, H), lambda i: (i, 0)),
                 pl.BlockSpec((8, H), lambda i: (0, 0))],
      out_shape=[jax.ShapeDtypeStruct((N, H), jnp.float32),
                 jax.ShapeDtypeStruct((8, H), jnp.float32)],
  )(*parts, Wa, ba.reshape(1, H), Wb, bb.reshape(1, H))
  return m, st


def _affine_from_stats(st, g, be):
  mean = st[0] / N
  var = st[1] / N - mean * mean
  scale = g / jnp.sqrt(var + 1e-5)
  shift = be - mean * scale
  return scale.reshape(1, H), shift.reshape(1, H)


def _affine_relu_split(m, scale, shift):
  def body(m_ref, sc_ref, sh_ref, lo_ref, hi_ref):
    y = jnp.maximum(m_ref[...] * sc_ref[...] + sh_ref[...], 0.0)
    lo_ref[...] = y[:, :H // 2]
    hi_ref[...] = y[:, H // 2:]

  return pl.pallas_call(
      body,
      grid=(N // BN,),
      in_specs=[pl.BlockSpec((BN, H), lambda i: (i, 0)),
                pl.BlockSpec((1, H), lambda i: (0, 0)),
                pl.BlockSpec((1, H), lambda i: (0, 0))],
      out_specs=[pl.BlockSpec((BN, H // 2), lambda i: (i, 0)),
                 pl.BlockSpec((BN, H // 2), lambda i: (i, 0))],
      out_shape=[jax.ShapeDtypeStruct((N, H // 2), jnp.float32),
                 jax.ShapeDtypeStruct((N, H // 2), jnp.float32)],
  )(m, scale, shift)


def _final_head(m, scale, shift, Wfc, bfc):
  def body(m_ref, sc_ref, sh_ref, W_ref, b_ref, o_ref):
    y = jnp.maximum(m_ref[...] * sc_ref[...] + sh_ref[...], 0.0)
    logits = jnp.dot(y, W_ref[...]) + b_ref[...]
    mx = jnp.max(logits, axis=1, keepdims=True)
    lse = jnp.log(jnp.sum(jnp.exp(logits - mx), axis=1, keepdims=True))
    o_ref[...] = logits - mx - lse

  return pl.pallas_call(
      body,
      grid=(N // BN,),
      in_specs=[pl.BlockSpec((BN, H), lambda i: (i, 0)),
                pl.BlockSpec((1, H), lambda i: (0, 0)),
                pl.BlockSpec((1, H), lambda i: (0, 0)),
                pl.BlockSpec((H, C), lambda i: (0, 0)),
                pl.BlockSpec((1, C), lambda i: (0, 0))],
      out_specs=pl.BlockSpec((BN, C), lambda i: (i, 0)),
      out_shape=jax.ShapeDtypeStruct((N, C), jnp.float32),
  )(m, scale, shift, Wfc, bfc.reshape(1, C))


def kernel(x, edge_index, W1a, b1a, W1b, b1b, g1, be1,
           W2a, b2a, W2b, b2b, g2, be2, Wfc, bfc):
  srcp = jnp.concatenate([edge_index[0], jnp.zeros((EPAD - E,), jnp.int32)])
  pad_dst = N + jnp.arange(EPAD - E, dtype=jnp.int32) % (NPAD - N)
  dstp = jnp.concatenate([edge_index[1], pad_dst])
  zrows = jnp.zeros((ROW_CHUNK, DC), jnp.float32)

  a10, a11 = _sc_edgesplit(x, srcp, dstp, zrows)
  m1, st1 = _gin_mlp(
      [x, a10, a11], lambda v: v[0] + v[1] + v[2],
      W1a, b1a, W1b, b1b, DIN)
  sc1, sh1 = _affine_from_stats(st1, g1, be1)
  y1lo, y1hi = _affine_relu_split(m1, sc1, sh1)

  a2lo, a2hi = _sc_colsplit(y1lo, y1hi, srcp, dstp, zrows)
  m2, st2 = _gin_mlp(
      [y1lo, y1hi, a2lo, a2hi],
      lambda v: jnp.concatenate([v[0] + v[2], v[1] + v[3]], axis=1),
      W2a, b2a, W2b, b2b, H)
  sc2, sh2 = _affine_from_stats(st2, g2, be2)

  return _final_head(m2, sc2, sh2, Wfc, bfc)


# speedup vs baseline: 1.8653x; 1.1580x over previous
import functools

import jax
import jax.numpy as jnp
from jax import lax
from jax.experimental import pallas as pl
from jax.experimental.pallas import tpu as pltpu
from jax.experimental.pallas import tpu_sc as plsc

N = 10000
E = 320000
DIN = 128
H = 256
C = 64

DC = 128

NPAD = 10240
ROWS_PER_TILE = NPAD // 16
ROW_CHUNK = 128
N_ROW_CHUNKS = ROWS_PER_TILE // ROW_CHUNK
ECHUNK = 128
NCHUNKS = 2528
EPAD = NCHUNKS * ECHUNK
NBUF = 2

BN = 2000


def _zero_acc(zrows, stage_v, acc, row0):
  pltpu.sync_copy(zrows, stage_v)
  for k in range(N_ROW_CHUNKS):
    pltpu.sync_copy(stage_v, acc.at[pl.ds(row0 + k * ROW_CHUNK, ROW_CHUNK)])


def _edge_loop(x_ref, srcp, dstp, srcs, dsts, rows, acc,
               isems, ssem, gsem, chunk0, nchunks):
  def start_idx(bs, bd, sem, j):
    e0 = (chunk0 + j) * ECHUNK
    pltpu.async_copy(srcp.at[pl.ds(e0, ECHUNK)], bs, sem)
    pltpu.async_copy(dstp.at[pl.ds(e0, ECHUNK)], bd, sem)

  def wait_idx(bs, bd, sem, j):
    e0 = (chunk0 + j) * ECHUNK
    pltpu.make_async_copy(srcp.at[pl.ds(e0, ECHUNK)], bs, sem).wait()
    pltpu.make_async_copy(dstp.at[pl.ds(e0, ECHUNK)], bd, sem).wait()

  def start_scatter(r, bd):
    pltpu.async_copy(rows[r], acc.at[bd], ssem[r], add=True)

  def wait_scatter(r, bd):
    pltpu.make_async_copy(rows[r], acc.at[bd], ssem[r]).wait()

  def step(u, j, guard_first):
    r, s = u % 2, u % 4
    wait_idx(srcs[s], dsts[s], isems[s], j)
    drain = functools.partial(wait_scatter, r, dsts[(u + 2) % 4])
    if guard_first:
      pl.when(j >= 2)(drain)
    else:
      drain()
    g = pltpu.async_copy(x_ref.at[srcs[s]], rows[r], gsem)
    pl.when(j + 2 < nchunks)(
        functools.partial(start_idx, srcs[(u + 2) % 4], dsts[(u + 2) % 4],
                          isems[(u + 2) % 4], j + 2))
    g.wait()
    start_scatter(r, dsts[s])

  start_idx(srcs[0], dsts[0], isems[0], 0)
  start_idx(srcs[1], dsts[1], isems[1], 1)

  def body(jo, carry):
    for u in range(4):
      step(u, jo * 4 + u, True)
    return carry

  lax.fori_loop(0, nchunks // 4, body, 0)
  for t in range(nchunks % 4):
    u = (nchunks // 4) * 4 + t
    step(u % 4, u, False)
  wait_scatter((nchunks - 2) % 2, dsts[(nchunks - 2) % 4])
  wait_scatter((nchunks - 1) % 2, dsts[(nchunks - 1) % 4])


def _write_acc(acc, stage_v, out_ref, row0):
  for k in range(N_ROW_CHUNKS):
    r = row0 + k * ROW_CHUNK
    pltpu.sync_copy(acc.at[pl.ds(r, ROW_CHUNK)], stage_v)
    pltpu.sync_copy(stage_v, out_ref.at[pl.ds(r, ROW_CHUNK)])


def _sc_scratch():
  return [
      [pltpu.VMEM((ECHUNK,), jnp.int32)] * 4,
      [pltpu.VMEM((ECHUNK,), jnp.int32)] * 4,
      [pltpu.VMEM((ECHUNK, DC), jnp.float32)] * 2,
      pltpu.VMEM_SHARED((NPAD, DC), jnp.float32),
      [pltpu.SemaphoreType.DMA] * 4,
      [pltpu.SemaphoreType.DMA] * 2,
      pltpu.SemaphoreType.DMA,
  ]


def _make_sc_scatter_edgesplit():
  mesh = plsc.VectorSubcoreMesh(core_axis_name="c", subcore_axis_name="s")

  @functools.partial(
      pl.kernel,
      out_type=(jax.ShapeDtypeStruct((NPAD, DC), jnp.float32),
                jax.ShapeDtypeStruct((NPAD, DC), jnp.float32)),
      mesh=mesh,
      scratch_types=_sc_scratch(),
  )
  def sc_kernel(x, srcp, dstp, zrows, out0, out1,
                srcs, dsts, rows, acc, isems, ssem, gsem):
    cid = lax.axis_index("c")
    sid = lax.axis_index("s")
    row0 = sid * ROWS_PER_TILE
    per_tile = NCHUNKS // 32

    _zero_acc(zrows, rows[0], acc, row0)
    plsc.subcore_barrier()
    chunk0 = (cid * 16 + sid) * per_tile
    _edge_loop(x, srcp, dstp, srcs, dsts, rows, acc,
               isems, ssem, gsem, chunk0, per_tile)
    plsc.subcore_barrier()
    pl.when(cid == 0)(lambda: _write_acc(acc, rows[0], out0, row0))
    pl.when(cid == 1)(lambda: _write_acc(acc, rows[0], out1, row0))

  return sc_kernel


def _make_sc_scatter_colsplit():
  mesh = plsc.VectorSubcoreMesh(core_axis_name="c", subcore_axis_name="s")

  @functools.partial(
      pl.kernel,
      out_type=(jax.ShapeDtypeStruct((NPAD, DC), jnp.float32),
                jax.ShapeDtypeStruct((NPAD, DC), jnp.float32)),
      mesh=mesh,
      scratch_types=_sc_scratch(),
  )
  def sc_kernel(xlo, xhi, srcp, dstp, zrows, outlo, outhi,
                srcs, dsts, rows, acc, isems, ssem, gsem):
    cid = lax.axis_index("c")
    sid = lax.axis_index("s")
    row0 = sid * ROWS_PER_TILE
    per_tile = NCHUNKS // 16

    def run(x_ref, out_ref):
      _zero_acc(zrows, rows[0], acc, row0)
      plsc.subcore_barrier()
      _edge_loop(x_ref, srcp, dstp, srcs, dsts, rows, acc,
                 isems, ssem, gsem, sid * per_tile, per_tile)
      plsc.subcore_barrier()
      _write_acc(acc, rows[0], out_ref, row0)

    pl.when(cid == 0)(lambda: run(xlo, outlo))
    pl.when(cid == 1)(lambda: run(xhi, outhi))

  return sc_kernel


_sc_edgesplit = _make_sc_scatter_edgesplit()
_sc_colsplit = _make_sc_scatter_colsplit()


def _gin_mlp(parts, combine, Wa, ba, Wb, bb, din):
  nin = len(parts)

  def body(*refs):
    Wa_ref, ba_ref, Wb_ref, bb_ref, m_ref, st_ref = refs[nin:]
    h = combine([r[...] for r in refs[:nin]])
    u = jnp.maximum(jnp.dot(h, Wa_ref[...]) + ba_ref[...], 0.0)
    m = jnp.dot(u, Wb_ref[...]) + bb_ref[...]
    m_ref[...] = m

    @pl.when(pl.program_id(0) == 0)
    def _():
      st_ref[...] = jnp.zeros_like(st_ref)

    st_ref[0:1, :] = st_ref[0:1, :] + jnp.sum(m, axis=0, keepdims=True)
    st_ref[1:2, :] = st_ref[1:2, :] + jnp.sum(m * m, axis=0, keepdims=True)

  in_specs = (
      [pl.BlockSpec((BN, p.shape[1]), lambda i: (i, 0)) for p in parts]
      + [pl.BlockSpec((din, H), lambda i: (0, 0)),
         pl.BlockSpec((1, H), lambda i: (0, 0)),
         pl.BlockSpec((H, H), lambda i: (0, 0)),
         pl.BlockSpec((1, H), lambda i: (0, 0))]
  )
  m, st = pl.pallas_call(
      body,
      grid=(N // BN,),
      in_specs=in_specs,
      out_specs=[pl.BlockSpec((BN, H), lambda i: (i, 0)),
                 pl.BlockSpec((8, H), lambda i: (0, 0))],
      out_shape=[jax.ShapeDtypeStruct((N, H), jnp.float32),
                 jax.ShapeDtypeStruct((8, H), jnp.float32)],
  )(*parts, Wa, ba.reshape(1, H), Wb, bb.reshape(1, H))
  return m, st


def _affine_from_stats(st, g, be):
  mean = st[0] / N
  var = st[1] / N - mean * mean
  scale = g / jnp.sqrt(var + 1e-5)
  shift = be - mean * scale
  return scale.reshape(1, H), shift.reshape(1, H)


def _affine_relu_split(m, scale, shift):
  def body(m_ref, sc_ref, sh_ref, lo_ref, hi_ref):
    y = jnp.maximum(m_ref[...] * sc_ref[...] + sh_ref[...], 0.0)
    lo_ref[...] = y[:, :H // 2]
    hi_ref[...] = y[:, H // 2:]

  return pl.pallas_call(
      body,
      grid=(N // BN,),
      in_specs=[pl.BlockSpec((BN, H), lambda i: (i, 0)),
                pl.BlockSpec((1, H), lambda i: (0, 0)),
                pl.BlockSpec((1, H), lambda i: (0, 0))],
      out_specs=[pl.BlockSpec((BN, H // 2), lambda i: (i, 0)),
                 pl.BlockSpec((BN, H // 2), lambda i: (i, 0))],
      out_shape=[jax.ShapeDtypeStruct((N, H // 2), jnp.float32),
                 jax.ShapeDtypeStruct((N, H // 2), jnp.float32)],
  )(m, scale, shift)


def _final_head(m, scale, shift, Wfc, bfc):
  def body(m_ref, sc_ref, sh_ref, W_ref, b_ref, o_ref):
    y = jnp.maximum(m_ref[...] * sc_ref[...] + sh_ref[...], 0.0)
    logits = jnp.dot(y, W_ref[...]) + b_ref[...]
    mx = jnp.max(logits, axis=1, keepdims=True)
    lse = jnp.log(jnp.sum(jnp.exp(logits - mx), axis=1, keepdims=True))
    o_ref[...] = logits - mx - lse

  return pl.pallas_call(
      body,
      grid=(N // BN,),
      in_specs=[pl.BlockSpec((BN, H), lambda i: (i, 0)),
                pl.BlockSpec((1, H), lambda i: (0, 0)),
                pl.BlockSpec((1, H), lambda i: (0, 0)),
                pl.BlockSpec((H, C), lambda i: (0, 0)),
                pl.BlockSpec((1, C), lambda i: (0, 0))],
      out_specs=pl.BlockSpec((BN, C), lambda i: (i, 0)),
      out_shape=jax.ShapeDtypeStruct((N, C), jnp.float32),
  )(m, scale, shift, Wfc, bfc.reshape(1, C))


def kernel(x, edge_index, W1a, b1a, W1b, b1b, g1, be1,
           W2a, b2a, W2b, b2b, g2, be2, Wfc, bfc):
  srcp = jnp.concatenate([edge_index[0], jnp.zeros((EPAD - E,), jnp.int32)])
  pad_dst = N + jnp.arange(EPAD - E, dtype=jnp.int32) % (NPAD - N)
  dstp = jnp.concatenate([edge_index[1], pad_dst])
  zrows = jnp.zeros((ROW_CHUNK, DC), jnp.float32)

  a10, a11 = _sc_edgesplit(x, srcp, dstp, zrows)
  m1, st1 = _gin_mlp(
      [x, a10, a11], lambda v: v[0] + v[1] + v[2],
      W1a, b1a, W1b, b1b, DIN)
  sc1, sh1 = _affine_from_stats(st1, g1, be1)
  y1lo, y1hi = _affine_relu_split(m1, sc1, sh1)

  a2lo, a2hi = _sc_colsplit(y1lo, y1hi, srcp, dstp, zrows)
  m2, st2 = _gin_mlp(
      [y1lo, y1hi, a2lo, a2hi],
      lambda v: jnp.concatenate([v[0] + v[2], v[1] + v[3]], axis=1),
      W2a, b2a, W2b, b2b, H)
  sc2, sh2 = _affine_from_stats(st2, g2, be2)

  return _final_head(m2, sc2, sh2, Wfc, bfc)
